# baseline (speedup 1.0000x reference)
import functools

import jax
import jax.numpy as jnp
from jax import lax
from jax.experimental import pallas as pl
from jax.experimental.pallas import tpu as pltpu
from jax.experimental.pallas import tpu_sc as plsc

B = 16384
D = 32
NU = 1000000
NQ = 100000
TBLK = 32768

_info = plsc.get_sparse_core_info()
_NC, _NS = _info.num_cores, _info.num_subcores
_NW = _NC * _NS
_BPW = B // _NW
_CHUNK = 128
_NCH = _BPW // _CHUNK


def _project_tc_q(tab_t, W, b2d):
    n = tab_t.shape[1]

    def body(t_ref, w_ref, b_ref, o_ref):
        w = w_ref[...]
        wd = w[0:1, D:] - w[1:2, D:]
        s = lax.dot_general(wd, t_ref[...], (((1,), (0,)), ((), ())),
                            preferred_element_type=jnp.float32)
        bb = b_ref[...]
        o_ref[...] = s[0] + (bb[0, 0] - bb[0, 1])

    return pl.pallas_call(
        body,
        grid=(pl.cdiv(n, TBLK),),
        in_specs=[
            pl.BlockSpec((D, TBLK), lambda i: (0, i)),
            pl.BlockSpec((2, 2 * D), lambda i: (0, 0)),
            pl.BlockSpec((1, 2), lambda i: (0, 0)),
        ],
        out_specs=pl.BlockSpec((TBLK,), lambda i: (i,)),
        out_shape=jax.ShapeDtypeStruct((n,), jnp.float32),
    )(tab_t, W, b2d)


def _project_tc_u(tab_t, W):
    n = tab_t.shape[1]

    def body(t_ref, w_ref, o_ref):
        w = w_ref[...]
        wd = w[0:1, :D] - w[1:2, :D]
        s = lax.dot_general(wd, t_ref[...], (((1,), (0,)), ((), ())),
                            preferred_element_type=jnp.float32)
        o_ref[...] = s[0]

    return pl.pallas_call(
        body,
        grid=(pl.cdiv(n, TBLK),),
        in_specs=[
            pl.BlockSpec((D, TBLK), lambda i: (0, i)),
            pl.BlockSpec((2, 2 * D), lambda i: (0, 0)),
        ],
        out_specs=pl.BlockSpec((TBLK,), lambda i: (i,)),
        out_shape=jax.ShapeDtypeStruct((n,), jnp.float32),
    )(tab_t, W)


def _gather_sigmoid_sc(s_u, s_q, uids2d, qids2d):
    mesh = plsc.VectorSubcoreMesh(core_axis_name="c", subcore_axis_name="s")

    @functools.partial(
        pl.kernel,
        mesh=mesh,
        out_type=[
            jax.ShapeDtypeStruct((B,), jnp.float32),
            jax.ShapeDtypeStruct((B,), jnp.float32),
        ],
        scratch_types=[
            pltpu.VMEM((_NCH, _CHUNK), jnp.int32),
            pltpu.VMEM((_NCH, _CHUNK), jnp.int32),
            pltpu.VMEM((_BPW,), jnp.float32),
            pltpu.VMEM((_BPW,), jnp.float32),
            pltpu.VMEM((_BPW,), jnp.float32),
            pltpu.VMEM((_BPW,), jnp.float32),
            pltpu.SemaphoreType.DMA,
        ],
    )
    def body(su_hbm, sq_hbm, uids, qids, p0_hbm, p1_hbm,
             uidx, qidx, su_v, sq_v, p0_v, p1_v, sem):
        wid = lax.axis_index("s") * _NC + lax.axis_index("c")
        base = wid * _BPW
        cu = pltpu.async_copy(uids.at[pl.ds(wid * _NCH, _NCH)], uidx, sem)
        cq = pltpu.async_copy(qids.at[pl.ds(wid * _NCH, _NCH)], qidx, sem)
        cu.wait()
        cq.wait()
        copies = []
        for j in range(_NCH):
            sl = pl.ds(j * _CHUNK, _CHUNK)
            copies.append(pltpu.async_copy(
                su_hbm.at[uidx.at[j]], su_v.at[sl], sem))
            copies.append(pltpu.async_copy(
                sq_hbm.at[qidx.at[j]], sq_v.at[sl], sem))
        for c in copies:
            c.wait()
        for k in range(_BPW // 16):
            sl = pl.ds(k * 16, 16)
            ld = su_v[sl] + sq_v[sl]
            p0 = 1.0 / (1.0 + jnp.exp(-ld))
            p0_v[sl] = p0
            p1_v[sl] = 1.0 - p0
        pltpu.sync_copy(p0_v, p0_hbm.at[pl.ds(base, _BPW)])
        pltpu.sync_copy(p1_v, p1_hbm.at[pl.ds(base, _BPW)])

    return body(s_u, s_q, uids2d, qids2d)


def kernel(user_ids, question_ids, user_table, question_table, W, b):
    s_u = _project_tc_u(user_table.T, W)
    s_q = _project_tc_q(question_table.T, W, b.reshape(1, 2))
    uids2d = user_ids.astype(jnp.int32).reshape(B // _CHUNK, _CHUNK)
    qids2d = question_ids.astype(jnp.int32).reshape(B // _CHUNK, _CHUNK)
    p0, p1 = _gather_sigmoid_sc(s_u, s_q, uids2d, qids2d)
    return jnp.stack([p0, p1], axis=-1)


# speedup vs baseline: 1.0809x; 1.0809x over previous
import functools

import jax
import jax.numpy as jnp
from jax import lax
from jax.experimental import pallas as pl
from jax.experimental.pallas import tpu as pltpu
from jax.experimental.pallas import tpu_sc as plsc

B = 16384
D = 32
NU = 1000000
NQ = 100000
TBLK = 65536

_info = plsc.get_sparse_core_info()
_NC, _NS = _info.num_cores, _info.num_subcores
_NW = _NC * _NS
_BPW = B // _NW
_CHUNK = 128
_NCH = _BPW // _CHUNK


def _project_tc_q(tab_t, W, b2d):
    n = tab_t.shape[1]

    def body(t_ref, w_ref, b_ref, o_ref):
        w = w_ref[...]
        wd = w[0:1, D:] - w[1:2, D:]
        s = lax.dot_general(wd, t_ref[...], (((1,), (0,)), ((), ())),
                            preferred_element_type=jnp.float32)
        bb = b_ref[...]
        o_ref[...] = s[0] + (bb[0, 0] - bb[0, 1])

    return pl.pallas_call(
        body,
        grid=(pl.cdiv(n, TBLK),),
        in_specs=[
            pl.BlockSpec((D, TBLK), lambda i: (0, i)),
            pl.BlockSpec((2, 2 * D), lambda i: (0, 0)),
            pl.BlockSpec((1, 2), lambda i: (0, 0)),
        ],
        out_specs=pl.BlockSpec((TBLK,), lambda i: (i,)),
        out_shape=jax.ShapeDtypeStruct((n,), jnp.float32),
    )(tab_t, W, b2d)


def _project_tc_u(tab_t, W):
    n = tab_t.shape[1]

    def body(t_ref, w_ref, o_ref):
        w = w_ref[...]
        wd = w[0:1, :D] - w[1:2, :D]
        s = lax.dot_general(wd, t_ref[...], (((1,), (0,)), ((), ())),
                            preferred_element_type=jnp.float32)
        o_ref[...] = s[0]

    return pl.pallas_call(
        body,
        grid=(pl.cdiv(n, TBLK),),
        in_specs=[
            pl.BlockSpec((D, TBLK), lambda i: (0, i)),
            pl.BlockSpec((2, 2 * D), lambda i: (0, 0)),
        ],
        out_specs=pl.BlockSpec((TBLK,), lambda i: (i,)),
        out_shape=jax.ShapeDtypeStruct((n,), jnp.float32),
    )(tab_t, W)


def _gather_sigmoid_sc(s_u, s_q, uids2d, qids2d):
    mesh = plsc.VectorSubcoreMesh(core_axis_name="c", subcore_axis_name="s")

    @functools.partial(
        pl.kernel,
        mesh=mesh,
        out_type=[
            jax.ShapeDtypeStruct((B,), jnp.float32),
            jax.ShapeDtypeStruct((B,), jnp.float32),
        ],
        scratch_types=[
            pltpu.VMEM((_NCH, _CHUNK), jnp.int32),
            pltpu.VMEM((_NCH, _CHUNK), jnp.int32),
            pltpu.VMEM((_BPW,), jnp.float32),
            pltpu.VMEM((_BPW,), jnp.float32),
            pltpu.VMEM((_BPW,), jnp.float32),
            pltpu.VMEM((_BPW,), jnp.float32),
            pltpu.SemaphoreType.DMA,
        ],
    )
    def body(su_hbm, sq_hbm, uids, qids, p0_hbm, p1_hbm,
             uidx, qidx, su_v, sq_v, p0_v, p1_v, sem):
        wid = lax.axis_index("s") * _NC + lax.axis_index("c")
        base = wid * _BPW
        cu = pltpu.async_copy(uids.at[pl.ds(wid * _NCH, _NCH)], uidx, sem)
        cq = pltpu.async_copy(qids.at[pl.ds(wid * _NCH, _NCH)], qidx, sem)
        cu.wait()
        cq.wait()
        copies = []
        for j in range(_NCH):
            sl = pl.ds(j * _CHUNK, _CHUNK)
            copies.append(pltpu.async_copy(
                su_hbm.at[uidx.at[j]], su_v.at[sl], sem))
            copies.append(pltpu.async_copy(
                sq_hbm.at[qidx.at[j]], sq_v.at[sl], sem))
        for c in copies:
            c.wait()
        for k in range(_BPW // 16):
            sl = pl.ds(k * 16, 16)
            ld = su_v[sl] + sq_v[sl]
            p0 = 1.0 / (1.0 + jnp.exp(-ld))
            p0_v[sl] = p0
            p1_v[sl] = 1.0 - p0
        pltpu.sync_copy(p0_v, p0_hbm.at[pl.ds(base, _BPW)])
        pltpu.sync_copy(p1_v, p1_hbm.at[pl.ds(base, _BPW)])

    return body(s_u, s_q, uids2d, qids2d)


def kernel(user_ids, question_ids, user_table, question_table, W, b):
    s_u = _project_tc_u(user_table.T, W)
    s_q = _project_tc_q(question_table.T, W, b.reshape(1, 2))
    uids2d = user_ids.astype(jnp.int32).reshape(B // _CHUNK, _CHUNK)
    qids2d = question_ids.astype(jnp.int32).reshape(B // _CHUNK, _CHUNK)
    p0, p1 = _gather_sigmoid_sc(s_u, s_q, uids2d, qids2d)
    return jnp.stack([p0, p1], axis=-1)
